# baseline (speedup 1.0000x reference)
import functools

import jax
import jax.numpy as jnp
from jax import lax
from jax.experimental import pallas as pl
from jax.experimental.pallas import tpu as pltpu
from jax.experimental.pallas import tpu_sc as plsc

M = 16384
N = 16384
NC = 2
NS = 16
NW = NC * NS
C = 2
L = 16


def _sc_mv(sc_rows: int, row_base: int):
    R = sc_rows // NW
    NCH = R // C
    mesh = plsc.VectorSubcoreMesh(core_axis_name="c", subcore_axis_name="s")

    @functools.partial(
        pl.kernel,
        out_type=jax.ShapeDtypeStruct((sc_rows,), jnp.float32),
        mesh=mesh,
        compiler_params=pltpu.CompilerParams(needs_layout_passes=False),
        scratch_types=[
            pltpu.VMEM((N,), jnp.float32),
            pltpu.VMEM((C, N), jnp.float32),
            pltpu.VMEM((C, N), jnp.float32),
            pltpu.VMEM((R,), jnp.float32),
            pltpu.SemaphoreType.DMA,
            pltpu.SemaphoreType.DMA,
        ],
    )
    def k(A_hbm, x_hbm, y_hbm, x_v, buf0, buf1, out_v, sem0, sem1):
        wid = lax.axis_index("s") * NC + lax.axis_index("c")
        out_base = wid * R
        base = row_base + out_base
        pltpu.sync_copy(x_hbm, x_v)

        def chunk_src(kk):
            return A_hbm.at[pl.ds(base + kk * C, C)]

        def compute(buf, kk):
            zero = jnp.zeros((L,), jnp.float32)

            def jbody(j, carry):
                a0, a1 = carry
                xv = x_v[pl.ds(j * L, L)]
                a0 = a0 + buf[0, pl.ds(j * L, L)] * xv
                a1 = a1 + buf[1, pl.ds(j * L, L)] * xv
                return (a0, a1)

            acc0, acc1 = lax.fori_loop(0, N // L, jbody, (zero, zero),
                                       unroll=8)
            lanes = lax.iota(jnp.int32, L)
            lane0 = lanes == 0
            for sub, acc in ((0, acc0), (1, acc1)):
                for sh in (8, 4, 2, 1):
                    acc = acc + acc.at[lanes ^ sh].get(
                        mode="promise_in_bounds")
                idx = jnp.full((L,), kk * C + sub, jnp.int32)
                plsc.store_scatter(out_v, [idx], acc, mask=lane0)

        pltpu.async_copy(chunk_src(0), buf0, sem0)
        pltpu.async_copy(chunk_src(1), buf1, sem1)

        def gbody(g, carry):
            k0 = 2 * g
            k1 = 2 * g + 1
            pltpu.make_async_copy(chunk_src(k0), buf0, sem0).wait()
            compute(buf0, k0)

            @pl.when(k0 + 2 < NCH)
            def _():
                pltpu.async_copy(chunk_src(k0 + 2), buf0, sem0)

            pltpu.make_async_copy(chunk_src(k1), buf1, sem1).wait()
            compute(buf1, k1)

            @pl.when(k1 + 2 < NCH)
            def _():
                pltpu.async_copy(chunk_src(k1 + 2), buf1, sem1)

            return carry

        lax.fori_loop(0, NCH // 2, gbody, 0)
        pltpu.sync_copy(out_v, y_hbm.at[pl.ds(out_base, R)])

    return k


def _tc_mv_body(a_ref, x_ref, o_ref):
    o_ref[...] = jnp.sum(a_ref[...] * x_ref[...], axis=1)


def _tc_mv(tc_rows: int, bm: int):
    def call(A, x):
        return pl.pallas_call(
            _tc_mv_body,
            grid=(tc_rows // bm,),
            in_specs=[
                pl.BlockSpec((bm, N), lambda i: (i, 0)),
                pl.BlockSpec((1, N), lambda i: (0, 0)),
            ],
            out_specs=pl.BlockSpec((bm,), lambda i: (i,)),
            out_shape=jax.ShapeDtypeStruct((tc_rows,), jnp.float32),
        )(A, x.reshape(1, N))
    return call


M_TC = 14336
M_SC = M - M_TC


def kernel(A, x):
    y_tc = _tc_mv(M_TC, 256)(A, x)
    if M_SC == 0:
        return y_tc
    y_sc = _sc_mv(M_SC, M_TC)(A, x)
    return jnp.concatenate([y_tc, y_sc])


# speedup vs baseline: 1.0030x; 1.0030x over previous
import functools

import jax
import jax.numpy as jnp
from jax import lax
from jax.experimental import pallas as pl
from jax.experimental.pallas import tpu as pltpu
from jax.experimental.pallas import tpu_sc as plsc

M = 16384
N = 16384
NC = 2
NS = 16
NW = NC * NS
C = 2
L = 16


def _sc_mv(sc_rows: int, row_base: int):
    R = sc_rows // NW
    NCH = R // C
    mesh = plsc.VectorSubcoreMesh(core_axis_name="c", subcore_axis_name="s")

    @functools.partial(
        pl.kernel,
        out_type=jax.ShapeDtypeStruct((sc_rows,), jnp.float32),
        mesh=mesh,
        compiler_params=pltpu.CompilerParams(needs_layout_passes=False),
        scratch_types=[
            pltpu.VMEM((N,), jnp.float32),
            pltpu.VMEM((C, N), jnp.float32),
            pltpu.VMEM((C, N), jnp.float32),
            pltpu.VMEM((R,), jnp.float32),
            pltpu.SemaphoreType.DMA,
            pltpu.SemaphoreType.DMA,
        ],
    )
    def k(A_hbm, x_hbm, y_hbm, x_v, buf0, buf1, out_v, sem0, sem1):
        wid = lax.axis_index("s") * NC + lax.axis_index("c")
        out_base = wid * R
        base = row_base + out_base
        pltpu.sync_copy(x_hbm, x_v)

        def chunk_src(kk):
            return A_hbm.at[pl.ds(base + kk * C, C)]

        def compute(buf, kk):
            zero = jnp.zeros((L,), jnp.float32)

            def jbody(j, carry):
                a0, a1 = carry
                xv = x_v[pl.ds(j * L, L)]
                a0 = a0 + buf[0, pl.ds(j * L, L)] * xv
                a1 = a1 + buf[1, pl.ds(j * L, L)] * xv
                return (a0, a1)

            acc0, acc1 = lax.fori_loop(0, N // L, jbody, (zero, zero),
                                       unroll=8)
            lanes = lax.iota(jnp.int32, L)
            lane0 = lanes == 0
            for sub, acc in ((0, acc0), (1, acc1)):
                for sh in (8, 4, 2, 1):
                    acc = acc + acc.at[lanes ^ sh].get(
                        mode="promise_in_bounds")
                idx = jnp.full((L,), kk * C + sub, jnp.int32)
                plsc.store_scatter(out_v, [idx], acc, mask=lane0)

        pltpu.async_copy(chunk_src(0), buf0, sem0)
        pltpu.async_copy(chunk_src(1), buf1, sem1)

        def gbody(g, carry):
            k0 = 2 * g
            k1 = 2 * g + 1
            pltpu.make_async_copy(chunk_src(k0), buf0, sem0).wait()
            compute(buf0, k0)

            @pl.when(k0 + 2 < NCH)
            def _():
                pltpu.async_copy(chunk_src(k0 + 2), buf0, sem0)

            pltpu.make_async_copy(chunk_src(k1), buf1, sem1).wait()
            compute(buf1, k1)

            @pl.when(k1 + 2 < NCH)
            def _():
                pltpu.async_copy(chunk_src(k1 + 2), buf1, sem1)

            return carry

        lax.fori_loop(0, NCH // 2, gbody, 0)
        pltpu.sync_copy(out_v, y_hbm.at[pl.ds(out_base, R)])

    return k


def _tc_mv_body(a_ref, x_ref, o_ref):
    o_ref[...] = jnp.sum(a_ref[...] * x_ref[...], axis=1)


def _tc_mv(tc_rows: int, bm: int):
    def call(A, x):
        return pl.pallas_call(
            _tc_mv_body,
            grid=(tc_rows // bm,),
            in_specs=[
                pl.BlockSpec((bm, N), lambda i: (i, 0)),
                pl.BlockSpec((1, N), lambda i: (0, 0)),
            ],
            out_specs=pl.BlockSpec((bm,), lambda i: (i,)),
            out_shape=jax.ShapeDtypeStruct((tc_rows,), jnp.float32),
        )(A, x.reshape(1, N))
    return call


M_TC = 15872
M_SC = M - M_TC


def kernel(A, x):
    y_tc = _tc_mv(M_TC, 256)(A, x)
    if M_SC == 0:
        return y_tc
    y_sc = _sc_mv(M_SC, M_TC)(A, x)
    return jnp.concatenate([y_tc, y_sc])


# speedup vs baseline: 1.0677x; 1.0645x over previous
import functools

import jax
import jax.numpy as jnp
from jax import lax
from jax.experimental import pallas as pl
from jax.experimental.pallas import tpu as pltpu
from jax.experimental.pallas import tpu_sc as plsc

M = 16384
N = 16384
NC = 2
NS = 16
NW = NC * NS
C = 2
L = 16


def _sc_mv(sc_rows: int, row_base: int):
    R = sc_rows // NW
    NCH = R // C
    mesh = plsc.VectorSubcoreMesh(core_axis_name="c", subcore_axis_name="s")

    @functools.partial(
        pl.kernel,
        out_type=jax.ShapeDtypeStruct((sc_rows,), jnp.float32),
        mesh=mesh,
        compiler_params=pltpu.CompilerParams(needs_layout_passes=False),
        scratch_types=[
            pltpu.VMEM((N,), jnp.float32),
            pltpu.VMEM((C, N), jnp.float32),
            pltpu.VMEM((C, N), jnp.float32),
            pltpu.VMEM((R,), jnp.float32),
            pltpu.SemaphoreType.DMA,
            pltpu.SemaphoreType.DMA,
        ],
    )
    def k(A_hbm, x_hbm, y_hbm, x_v, buf0, buf1, out_v, sem0, sem1):
        wid = lax.axis_index("s") * NC + lax.axis_index("c")
        out_base = wid * R
        base = row_base + out_base
        pltpu.sync_copy(x_hbm, x_v)

        def chunk_src(kk):
            return A_hbm.at[pl.ds(base + kk * C, C)]

        def compute(buf, kk):
            zero = jnp.zeros((L,), jnp.float32)

            def jbody(j, carry):
                a0, a1 = carry
                xv = x_v[pl.ds(j * L, L)]
                a0 = a0 + buf[0, pl.ds(j * L, L)] * xv
                a1 = a1 + buf[1, pl.ds(j * L, L)] * xv
                return (a0, a1)

            acc0, acc1 = lax.fori_loop(0, N // L, jbody, (zero, zero),
                                       unroll=8)
            lanes = lax.iota(jnp.int32, L)
            lane0 = lanes == 0
            for sub, acc in ((0, acc0), (1, acc1)):
                for sh in (8, 4, 2, 1):
                    acc = acc + acc.at[lanes ^ sh].get(
                        mode="promise_in_bounds")
                idx = jnp.full((L,), kk * C + sub, jnp.int32)
                plsc.store_scatter(out_v, [idx], acc, mask=lane0)

        pltpu.async_copy(chunk_src(0), buf0, sem0)
        pltpu.async_copy(chunk_src(1), buf1, sem1)

        def gbody(g, carry):
            k0 = 2 * g
            k1 = 2 * g + 1
            pltpu.make_async_copy(chunk_src(k0), buf0, sem0).wait()
            compute(buf0, k0)

            @pl.when(k0 + 2 < NCH)
            def _():
                pltpu.async_copy(chunk_src(k0 + 2), buf0, sem0)

            pltpu.make_async_copy(chunk_src(k1), buf1, sem1).wait()
            compute(buf1, k1)

            @pl.when(k1 + 2 < NCH)
            def _():
                pltpu.async_copy(chunk_src(k1 + 2), buf1, sem1)

            return carry

        lax.fori_loop(0, NCH // 2, gbody, 0)
        pltpu.sync_copy(out_v, y_hbm.at[pl.ds(out_base, R)])

    return k


def _tc_mv_body(a_ref, x_ref, o_ref):
    o_ref[...] = jnp.sum(a_ref[...] * x_ref[...], axis=1)


def _tc_mv(tc_rows: int, bm: int):
    def call(A, x):
        return pl.pallas_call(
            _tc_mv_body,
            grid=(tc_rows // bm,),
            in_specs=[
                pl.BlockSpec((bm, N), lambda i: (i, 0)),
                pl.BlockSpec((1, N), lambda i: (0, 0)),
            ],
            out_specs=pl.BlockSpec((bm,), lambda i: (i,)),
            out_shape=jax.ShapeDtypeStruct((tc_rows,), jnp.float32),
        )(A, x.reshape(1, N))
    return call


M_TC = 16384
M_SC = M - M_TC


def kernel(A, x):
    y_tc = _tc_mv(M_TC, 128)(A, x)
    if M_SC == 0:
        return y_tc
    y_sc = _sc_mv(M_SC, M_TC)(A, x)
    return jnp.concatenate([y_tc, y_sc])


# speedup vs baseline: 1.0690x; 1.0012x over previous
import jax
import jax.numpy as jnp
from jax.experimental import pallas as pl

M = 16384
N = 16384
BM = 256


def _mv_body(a_ref, x_ref, o_ref):
    o_ref[...] = jnp.sum(a_ref[...] * x_ref[...], axis=1)


def kernel(A, x):
    return pl.pallas_call(
        _mv_body,
        grid=(M // BM,),
        in_specs=[
            pl.BlockSpec((BM, N), lambda i: (i, 0)),
            pl.BlockSpec((1, N), lambda i: (0, 0)),
        ],
        out_specs=pl.BlockSpec((BM,), lambda i: (i,)),
        out_shape=jax.ShapeDtypeStruct((M,), jnp.float32),
    )(A, x.reshape(1, N))
